# baseline (speedup 1.0000x reference)
import functools

import jax
import jax.numpy as jnp
from jax import lax
from jax.experimental import pallas as pl
from jax.experimental.pallas import tpu as pltpu
from jax.experimental.pallas import tpu_sc as plsc

N = 10000
E = 320000
D = 128
NPAD = 10240
RPT = NPAD // 16
K = 128
CH = 160
M = CH * K
EPAD = 16 * M
ZR = 128
G = 16
SUP = CH // G
SQ = 2
SR = RPT // SQ

_mesh = plsc.VectorSubcoreMesh(core_axis_name="c", subcore_axis_name="s")


@functools.partial(
    pl.kernel,
    mesh=_mesh,
    out_type=jax.ShapeDtypeStruct((2, NPAD, D), jnp.float32),
    scratch_types=[
        pltpu.VMEM((G, K), jnp.int32),
        pltpu.VMEM((G, K), jnp.int32),
        pltpu.VMEM((K, D), jnp.float32),
        pltpu.VMEM((K, D), jnp.float32),
        pltpu.SemaphoreType.DMA,
        pltpu.VMEM_SHARED((NPAD, D), jnp.float32),
    ],
)
def _sc_degrees(src_hbm, dst_hbm, out_hbm, src_v, dst_v, e0_v, e1_v, ss,
                deg_sh):
    c = lax.axis_index("c")
    s = lax.axis_index("s")
    lane = lax.iota(jnp.int32, 16)
    one = jnp.full((16,), 1.0, jnp.float32)
    zero = jnp.full((16,), 0.0, jnp.float32)
    e0 = jnp.where(lane == 0, one, zero)
    e1 = jnp.where(lane == 1, one, zero)

    @pl.loop(0, K)
    def _(r):
        @pl.loop(0, D // 16)
        def _(k):
            e0_v[r, pl.ds(k * 16, 16)] = zero
            e1_v[r, pl.ds(k * 16, 16)] = zero

    @pl.loop(0, RPT // ZR)
    def _(z):
        pltpu.sync_copy(e0_v, deg_sh.at[pl.ds(s * RPT + z * ZR, ZR)])

    @pl.loop(0, K)
    def _(r):
        e0_v[r, pl.ds(0, 16)] = e0
        e1_v[r, pl.ds(0, 16)] = e1

    def _wait_pair():
        pltpu.make_async_copy(e0_v, deg_sh.at[pl.ds(0, K)], ss).wait()
        pltpu.make_async_copy(e1_v, deg_sh.at[pl.ds(0, K)], ss).wait()

    def _run(g):
        plsc.subcore_barrier()

        @pl.loop(0, SUP)
        def _(u):
            pltpu.sync_copy(src_hbm.at[g].at[pl.ds(s * CH + u * G, G)], src_v)
            pltpu.sync_copy(dst_hbm.at[g].at[pl.ds(s * CH + u * G, G)], dst_v)
            pltpu.async_copy(e0_v, deg_sh.at[src_v.at[0]], ss, add=True)
            pltpu.async_copy(e1_v, deg_sh.at[dst_v.at[0]], ss, add=True)

            @pl.loop(1, G)
            def _(j):
                pltpu.async_copy(e0_v, deg_sh.at[src_v.at[j]], ss, add=True)
                pltpu.async_copy(e1_v, deg_sh.at[dst_v.at[j]], ss, add=True)
                _wait_pair()

            _wait_pair()

        plsc.subcore_barrier()
        pltpu.sync_copy(deg_sh.at[pl.ds(s * RPT, RPT)],
                        out_hbm.at[g].at[pl.ds(s * RPT, RPT)])

    @pl.when(c == 0)
    def _():
        _run(0)

    @pl.when(c == 1)
    def _():
        _run(1)


@functools.partial(
    pl.kernel,
    mesh=_mesh,
    out_type=jax.ShapeDtypeStruct((2, NPAD, D), jnp.float32),
    scratch_types=[
        pltpu.VMEM((G, K), jnp.int32),
        pltpu.VMEM((G, K), jnp.int32),
        pltpu.VMEM((K, D), jnp.float32),
        pltpu.VMEM((K, D), jnp.float32),
        pltpu.SemaphoreType.DMA,
        pltpu.SemaphoreType.DMA,
        pltpu.VMEM_SHARED((NPAD, D), jnp.float32),
    ],
)
def _sc_aggregate(table_hbm, src_hbm, dst_hbm, out_hbm, src_v, dst_v, rows_a,
                  rows_b, sga, sgb, agg_sh):
    c = lax.axis_index("c")
    s = lax.axis_index("s")
    zero = jnp.full((16,), 0.0, jnp.float32)

    @pl.loop(0, K)
    def _(r):
        @pl.loop(0, D // 16)
        def _(k):
            rows_a[r, pl.ds(k * 16, 16)] = zero

    @pl.loop(0, RPT // ZR)
    def _(z):
        pltpu.sync_copy(rows_a, agg_sh.at[pl.ds(s * RPT + z * ZR, ZR)])

    def _run(g):
        tbl = table_hbm.at[g]

        def _wait(buf, sem):
            pltpu.make_async_copy(tbl.at[pl.ds(0, K)], buf, sem).wait()

        plsc.subcore_barrier()

        @pl.loop(0, SUP)
        def _(u):
            pltpu.sync_copy(src_hbm.at[g].at[pl.ds(s * CH + u * G, G)], src_v)
            pltpu.sync_copy(dst_hbm.at[g].at[pl.ds(s * CH + u * G, G)], dst_v)
            pltpu.async_copy(tbl.at[src_v.at[0]], rows_a, sga)

            @pl.loop(0, G // 2)
            def _(jj):
                j = 2 * jj
                pltpu.async_copy(tbl.at[src_v.at[j + 1]], rows_b, sgb)
                _wait(rows_a, sga)
                pltpu.sync_copy(rows_a, agg_sh.at[dst_v.at[j]], add=True)

                @pl.when(j + 2 < G)
                def _():
                    pltpu.async_copy(tbl.at[src_v.at[j + 2]], rows_a, sga)

                _wait(rows_b, sgb)
                pltpu.sync_copy(rows_b, agg_sh.at[dst_v.at[j + 1]], add=True)

        plsc.subcore_barrier()
        pltpu.sync_copy(agg_sh.at[pl.ds(s * RPT, RPT)],
                        out_hbm.at[g].at[pl.ds(s * RPT, RPT)])

    @pl.when(c == 0)
    def _():
        _run(0)

    @pl.when(c == 1)
    def _():
        _run(1)


def _prep_body(x_ref, deg_ref, table_ref, sc_ref):
    deg = deg_ref[0]
    s_out = lax.rsqrt(jnp.maximum(deg[:, 0], 1.0))
    s_in = lax.rsqrt(jnp.maximum(deg[:, 1], 1.0))
    sc_ref[0, 0, :] = s_in
    sc_ref[0, 1, :] = s_out
    table_ref[0] = x_ref[0] * s_out[:, None]


def _tc_prep(x_pad, deg):
    return pl.pallas_call(
        _prep_body,
        grid=(2,),
        in_specs=[
            pl.BlockSpec((1, NPAD, D), lambda g: (g, 0, 0)),
            pl.BlockSpec((1, NPAD, D), lambda g: (g, 0, 0)),
        ],
        out_specs=[
            pl.BlockSpec((1, NPAD, D), lambda g: (g, 0, 0)),
            pl.BlockSpec((1, 2, NPAD), lambda g: (g, 0, 0)),
        ],
        out_shape=[
            jax.ShapeDtypeStruct((2, NPAD, D), jnp.float32),
            jax.ShapeDtypeStruct((2, 2, NPAD), jnp.float32),
        ],
    )(x_pad, deg)


BR = 1024
NB = NPAD // BR


def _layer_body(agg_ref, sc_ref, w_ref, b_ref, out_ref):
    a = agg_ref[0] * sc_ref[0, 0, :][:, None]
    h = jnp.dot(a, w_ref[...], preferred_element_type=jnp.float32,
                precision=lax.Precision.HIGHEST)
    h = jnp.maximum(h + b_ref[0][None, :], 0.0)
    out_ref[0] = h * sc_ref[0, 1, :][:, None]


def _tc_layer(agg, scales, w, b2d):
    return pl.pallas_call(
        _layer_body,
        grid=(2, NB),
        in_specs=[
            pl.BlockSpec((1, BR, D), lambda g, i: (g, i, 0)),
            pl.BlockSpec((1, 2, BR), lambda g, i: (g, 0, i)),
            pl.BlockSpec((D, D), lambda g, i: (0, 0)),
            pl.BlockSpec((1, D), lambda g, i: (0, 0)),
        ],
        out_specs=pl.BlockSpec((1, BR, D), lambda g, i: (g, i, 0)),
        out_shape=jax.ShapeDtypeStruct((2, NPAD, D), jnp.float32),
    )(agg, scales, w, b2d)


def _final_body(agg_ref, sc_ref, w_ref, b_ref, out_ref):
    i = pl.program_id(1)
    a = agg_ref[0] * sc_ref[0, 0, :][:, None]
    h = jnp.dot(a, w_ref[...], preferred_element_type=jnp.float32,
                precision=lax.Precision.HIGHEST)
    h = jnp.maximum(h + b_ref[0][None, :], 0.0)
    rows = i * BR + lax.broadcasted_iota(jnp.int32, (BR, D), 0)
    h = jnp.where(rows < N, h, 0.0)
    part = jnp.sum(h.reshape(BR // 8, 8, D), axis=0)

    @pl.when(i == 0)
    def _():
        out_ref[0] = part

    @pl.when(i != 0)
    def _():
        out_ref[0] += part


def _tc_final(agg, scales, w, b2d):
    return pl.pallas_call(
        _final_body,
        grid=(2, NB),
        in_specs=[
            pl.BlockSpec((1, BR, D), lambda g, i: (g, i, 0)),
            pl.BlockSpec((1, 2, BR), lambda g, i: (g, 0, i)),
            pl.BlockSpec((D, D), lambda g, i: (0, 0)),
            pl.BlockSpec((1, D), lambda g, i: (0, 0)),
        ],
        out_specs=pl.BlockSpec((1, 8, D), lambda g, i: (g, 0, 0)),
        out_shape=jax.ShapeDtypeStruct((2, 8, D), jnp.float32),
    )(agg, scales, w, b2d)


def _head_body(sums_ref, wf1a_ref, wf1b_ref, bf1_ref, wf2_ref, bf2_ref,
               out_ref):
    suml = jnp.sum(sums_ref[0], axis=0, keepdims=True)
    sumr = jnp.sum(sums_ref[1], axis=0, keepdims=True)
    h = (jnp.dot(suml, wf1a_ref[...], preferred_element_type=jnp.float32,
                 precision=lax.Precision.HIGHEST)
         + jnp.dot(sumr, wf1b_ref[...], preferred_element_type=jnp.float32,
                   precision=lax.Precision.HIGHEST)
         + bf1_ref[...])
    h = jnp.maximum(h, 0.0)
    out_ref[...] = (jnp.sum(h * wf2_ref[...], axis=1, keepdims=True)
                    + bf2_ref[...])


def _tc_head(sums, wf1a, wf1b, bf1_2d, wf2_t, bf2_2d):
    return pl.pallas_call(
        _head_body,
        out_shape=jax.ShapeDtypeStruct((1, 1), jnp.float32),
    )(sums, wf1a, wf1b, bf1_2d, wf2_t, bf2_2d)


def _pad_edges(idx):
    pad = EPAD - E
    fill = N + (jnp.arange(pad, dtype=jnp.int32) % (NPAD - N))
    return jnp.concatenate([idx.astype(jnp.int32), fill])


def kernel(xl, edge_index_l, xr, edge_index_r,
           W0, b0, W1, b1, W2, b2, Wf1, bf1, Wf2, bf2):
    src = jnp.stack([_pad_edges(edge_index_l[0]), _pad_edges(edge_index_r[0])])
    dst = jnp.stack([_pad_edges(edge_index_l[1]), _pad_edges(edge_index_r[1])])
    src = src.reshape(2, 16 * CH, K)
    dst = dst.reshape(2, 16 * CH, K)

    x_pad = jnp.stack([
        jnp.pad(xl, ((0, NPAD - N), (0, 0))),
        jnp.pad(xr, ((0, NPAD - N), (0, 0))),
    ])

    deg = _sc_degrees(src, dst)
    table, scales = _tc_prep(x_pad, deg)

    for w, b in ((W0, b0), (W1, b1)):
        agg = _sc_aggregate(table, src, dst)
        table = _tc_layer(agg, scales, w, b.reshape(1, D))

    agg = _sc_aggregate(table, src, dst)
    sums = _tc_final(agg, scales, W2, b2.reshape(1, D))

    return _tc_head(sums, Wf1[:D], Wf1[D:], bf1.reshape(1, D),
                    Wf2.reshape(1, D), bf2.reshape(1, 1))


# speedup vs baseline: 1.0450x; 1.0450x over previous
import functools

import jax
import jax.numpy as jnp
from jax import lax
from jax.experimental import pallas as pl
from jax.experimental.pallas import tpu as pltpu
from jax.experimental.pallas import tpu_sc as plsc

N = 10000
E = 320000
D = 128
NPAD = 10240
RPT = NPAD // 16
K = 128
CH = 160
M = CH * K
EPAD = 16 * M
ZR = 128
G = 32
SUP = CH // G
SQ = 2
SR = RPT // SQ

_mesh = plsc.VectorSubcoreMesh(core_axis_name="c", subcore_axis_name="s")


@functools.partial(
    pl.kernel,
    mesh=_mesh,
    out_type=jax.ShapeDtypeStruct((2, NPAD, D), jnp.float32),
    scratch_types=[
        pltpu.VMEM((G, K), jnp.int32),
        pltpu.VMEM((G, K), jnp.int32),
        pltpu.VMEM((K, D), jnp.float32),
        pltpu.VMEM((K, D), jnp.float32),
        pltpu.SemaphoreType.DMA,
        pltpu.VMEM_SHARED((NPAD, D), jnp.float32),
    ],
)
def _sc_degrees(src_hbm, dst_hbm, out_hbm, src_v, dst_v, e0_v, e1_v, ss,
                deg_sh):
    c = lax.axis_index("c")
    s = lax.axis_index("s")
    lane = lax.iota(jnp.int32, 16)
    one = jnp.full((16,), 1.0, jnp.float32)
    zero = jnp.full((16,), 0.0, jnp.float32)
    e0 = jnp.where(lane == 0, one, zero)
    e1 = jnp.where(lane == 1, one, zero)

    @pl.loop(0, K)
    def _(r):
        @pl.loop(0, D // 16)
        def _(k):
            e0_v[r, pl.ds(k * 16, 16)] = zero
            e1_v[r, pl.ds(k * 16, 16)] = zero

    @pl.loop(0, RPT // ZR)
    def _(z):
        pltpu.sync_copy(e0_v, deg_sh.at[pl.ds(s * RPT + z * ZR, ZR)])

    @pl.loop(0, K)
    def _(r):
        e0_v[r, pl.ds(0, 16)] = e0
        e1_v[r, pl.ds(0, 16)] = e1

    def _wait_pair():
        pltpu.make_async_copy(e0_v, deg_sh.at[pl.ds(0, K)], ss).wait()
        pltpu.make_async_copy(e1_v, deg_sh.at[pl.ds(0, K)], ss).wait()

    def _run(g):
        plsc.subcore_barrier()

        @pl.loop(0, SUP)
        def _(u):
            pltpu.sync_copy(src_hbm.at[g].at[pl.ds(s * CH + u * G, G)], src_v)
            pltpu.sync_copy(dst_hbm.at[g].at[pl.ds(s * CH + u * G, G)], dst_v)
            pltpu.async_copy(e0_v, deg_sh.at[src_v.at[0]], ss, add=True)
            pltpu.async_copy(e1_v, deg_sh.at[dst_v.at[0]], ss, add=True)

            @pl.loop(1, G)
            def _(j):
                pltpu.async_copy(e0_v, deg_sh.at[src_v.at[j]], ss, add=True)
                pltpu.async_copy(e1_v, deg_sh.at[dst_v.at[j]], ss, add=True)
                _wait_pair()

            _wait_pair()

        plsc.subcore_barrier()
        pltpu.sync_copy(deg_sh.at[pl.ds(s * RPT, RPT)],
                        out_hbm.at[g].at[pl.ds(s * RPT, RPT)])

    @pl.when(c == 0)
    def _():
        _run(0)

    @pl.when(c == 1)
    def _():
        _run(1)


@functools.partial(
    pl.kernel,
    mesh=_mesh,
    out_type=jax.ShapeDtypeStruct((2, NPAD, D), jnp.float32),
    scratch_types=[
        pltpu.VMEM((G, K), jnp.int32),
        pltpu.VMEM((G, K), jnp.int32),
        pltpu.VMEM((K, D), jnp.float32),
        pltpu.VMEM((K, D), jnp.float32),
        pltpu.SemaphoreType.DMA,
        pltpu.SemaphoreType.DMA,
        pltpu.VMEM_SHARED((NPAD, D), jnp.float32),
    ],
)
def _sc_aggregate(table_hbm, src_hbm, dst_hbm, out_hbm, src_v, dst_v, rows_a,
                  rows_b, sga, sgb, agg_sh):
    c = lax.axis_index("c")
    s = lax.axis_index("s")
    zero = jnp.full((16,), 0.0, jnp.float32)

    @pl.loop(0, K)
    def _(r):
        @pl.loop(0, D // 16)
        def _(k):
            rows_a[r, pl.ds(k * 16, 16)] = zero

    @pl.loop(0, RPT // ZR)
    def _(z):
        pltpu.sync_copy(rows_a, agg_sh.at[pl.ds(s * RPT + z * ZR, ZR)])

    def _run(g):
        tbl = table_hbm.at[g]

        def _wait(buf, sem):
            pltpu.make_async_copy(tbl.at[pl.ds(0, K)], buf, sem).wait()

        plsc.subcore_barrier()

        @pl.loop(0, SUP)
        def _(u):
            pltpu.sync_copy(src_hbm.at[g].at[pl.ds(s * CH + u * G, G)], src_v)
            pltpu.sync_copy(dst_hbm.at[g].at[pl.ds(s * CH + u * G, G)], dst_v)
            pltpu.async_copy(tbl.at[src_v.at[0]], rows_a, sga)

            @pl.loop(0, G // 2)
            def _(jj):
                j = 2 * jj
                pltpu.async_copy(tbl.at[src_v.at[j + 1]], rows_b, sgb)
                _wait(rows_a, sga)
                pltpu.sync_copy(rows_a, agg_sh.at[dst_v.at[j]], add=True)

                @pl.when(j + 2 < G)
                def _():
                    pltpu.async_copy(tbl.at[src_v.at[j + 2]], rows_a, sga)

                _wait(rows_b, sgb)
                pltpu.sync_copy(rows_b, agg_sh.at[dst_v.at[j + 1]], add=True)

        plsc.subcore_barrier()
        pltpu.sync_copy(agg_sh.at[pl.ds(s * RPT, RPT)],
                        out_hbm.at[g].at[pl.ds(s * RPT, RPT)])

    @pl.when(c == 0)
    def _():
        _run(0)

    @pl.when(c == 1)
    def _():
        _run(1)


def _prep_body(x_ref, deg_ref, table_ref, sc_ref):
    deg = deg_ref[0]
    s_out = lax.rsqrt(jnp.maximum(deg[:, 0], 1.0))
    s_in = lax.rsqrt(jnp.maximum(deg[:, 1], 1.0))
    sc_ref[0, 0, :] = s_in
    sc_ref[0, 1, :] = s_out
    table_ref[0] = x_ref[0] * s_out[:, None]


def _tc_prep(x_pad, deg):
    return pl.pallas_call(
        _prep_body,
        grid=(2,),
        in_specs=[
            pl.BlockSpec((1, NPAD, D), lambda g: (g, 0, 0)),
            pl.BlockSpec((1, NPAD, D), lambda g: (g, 0, 0)),
        ],
        out_specs=[
            pl.BlockSpec((1, NPAD, D), lambda g: (g, 0, 0)),
            pl.BlockSpec((1, 2, NPAD), lambda g: (g, 0, 0)),
        ],
        out_shape=[
            jax.ShapeDtypeStruct((2, NPAD, D), jnp.float32),
            jax.ShapeDtypeStruct((2, 2, NPAD), jnp.float32),
        ],
    )(x_pad, deg)


BR = 1024
NB = NPAD // BR


def _layer_body(agg_ref, sc_ref, w_ref, b_ref, out_ref):
    a = agg_ref[0] * sc_ref[0, 0, :][:, None]
    h = jnp.dot(a, w_ref[...], preferred_element_type=jnp.float32,
                precision=lax.Precision.HIGHEST)
    h = jnp.maximum(h + b_ref[0][None, :], 0.0)
    out_ref[0] = h * sc_ref[0, 1, :][:, None]


def _tc_layer(agg, scales, w, b2d):
    return pl.pallas_call(
        _layer_body,
        grid=(2, NB),
        in_specs=[
            pl.BlockSpec((1, BR, D), lambda g, i: (g, i, 0)),
            pl.BlockSpec((1, 2, BR), lambda g, i: (g, 0, i)),
            pl.BlockSpec((D, D), lambda g, i: (0, 0)),
            pl.BlockSpec((1, D), lambda g, i: (0, 0)),
        ],
        out_specs=pl.BlockSpec((1, BR, D), lambda g, i: (g, i, 0)),
        out_shape=jax.ShapeDtypeStruct((2, NPAD, D), jnp.float32),
    )(agg, scales, w, b2d)


def _final_body(agg_ref, sc_ref, w_ref, b_ref, out_ref):
    i = pl.program_id(1)
    a = agg_ref[0] * sc_ref[0, 0, :][:, None]
    h = jnp.dot(a, w_ref[...], preferred_element_type=jnp.float32,
                precision=lax.Precision.HIGHEST)
    h = jnp.maximum(h + b_ref[0][None, :], 0.0)
    rows = i * BR + lax.broadcasted_iota(jnp.int32, (BR, D), 0)
    h = jnp.where(rows < N, h, 0.0)
    part = jnp.sum(h.reshape(BR // 8, 8, D), axis=0)

    @pl.when(i == 0)
    def _():
        out_ref[0] = part

    @pl.when(i != 0)
    def _():
        out_ref[0] += part


def _tc_final(agg, scales, w, b2d):
    return pl.pallas_call(
        _final_body,
        grid=(2, NB),
        in_specs=[
            pl.BlockSpec((1, BR, D), lambda g, i: (g, i, 0)),
            pl.BlockSpec((1, 2, BR), lambda g, i: (g, 0, i)),
            pl.BlockSpec((D, D), lambda g, i: (0, 0)),
            pl.BlockSpec((1, D), lambda g, i: (0, 0)),
        ],
        out_specs=pl.BlockSpec((1, 8, D), lambda g, i: (g, 0, 0)),
        out_shape=jax.ShapeDtypeStruct((2, 8, D), jnp.float32),
    )(agg, scales, w, b2d)


def _head_body(sums_ref, wf1a_ref, wf1b_ref, bf1_ref, wf2_ref, bf2_ref,
               out_ref):
    suml = jnp.sum(sums_ref[0], axis=0, keepdims=True)
    sumr = jnp.sum(sums_ref[1], axis=0, keepdims=True)
    h = (jnp.dot(suml, wf1a_ref[...], preferred_element_type=jnp.float32,
                 precision=lax.Precision.HIGHEST)
         + jnp.dot(sumr, wf1b_ref[...], preferred_element_type=jnp.float32,
                   precision=lax.Precision.HIGHEST)
         + bf1_ref[...])
    h = jnp.maximum(h, 0.0)
    out_ref[...] = (jnp.sum(h * wf2_ref[...], axis=1, keepdims=True)
                    + bf2_ref[...])


def _tc_head(sums, wf1a, wf1b, bf1_2d, wf2_t, bf2_2d):
    return pl.pallas_call(
        _head_body,
        out_shape=jax.ShapeDtypeStruct((1, 1), jnp.float32),
    )(sums, wf1a, wf1b, bf1_2d, wf2_t, bf2_2d)


def _pad_edges(idx):
    pad = EPAD - E
    fill = N + (jnp.arange(pad, dtype=jnp.int32) % (NPAD - N))
    return jnp.concatenate([idx.astype(jnp.int32), fill])


def kernel(xl, edge_index_l, xr, edge_index_r,
           W0, b0, W1, b1, W2, b2, Wf1, bf1, Wf2, bf2):
    src = jnp.stack([_pad_edges(edge_index_l[0]), _pad_edges(edge_index_r[0])])
    dst = jnp.stack([_pad_edges(edge_index_l[1]), _pad_edges(edge_index_r[1])])
    src = src.reshape(2, 16 * CH, K)
    dst = dst.reshape(2, 16 * CH, K)

    x_pad = jnp.stack([
        jnp.pad(xl, ((0, NPAD - N), (0, 0))),
        jnp.pad(xr, ((0, NPAD - N), (0, 0))),
    ])

    deg = _sc_degrees(src, dst)
    table, scales = _tc_prep(x_pad, deg)

    for w, b in ((W0, b0), (W1, b1)):
        agg = _sc_aggregate(table, src, dst)
        table = _tc_layer(agg, scales, w, b.reshape(1, D))

    agg = _sc_aggregate(table, src, dst)
    sums = _tc_final(agg, scales, W2, b2.reshape(1, D))

    return _tc_head(sums, Wf1[:D], Wf1[D:], bf1.reshape(1, D),
                    Wf2.reshape(1, D), bf2.reshape(1, 1))


# speedup vs baseline: 1.0591x; 1.0135x over previous
import functools

import jax
import jax.numpy as jnp
from jax import lax
from jax.experimental import pallas as pl
from jax.experimental.pallas import tpu as pltpu
from jax.experimental.pallas import tpu_sc as plsc

N = 10000
E = 320000
D = 128
NPAD = 10240
RPT = NPAD // 16
K = 128
CH = 80
EPAD = 32 * CH * K
ZR = 128
G = 40
SUP = CH // G

_mesh = plsc.VectorSubcoreMesh(core_axis_name="c", subcore_axis_name="s")


@functools.partial(
    pl.kernel,
    mesh=_mesh,
    out_type=jax.ShapeDtypeStruct((2, NPAD, D), jnp.float32),
    scratch_types=[
        pltpu.VMEM((G, K), jnp.int32),
        pltpu.VMEM((G, K), jnp.int32),
        pltpu.VMEM((K, D), jnp.float32),
        pltpu.VMEM((K, D), jnp.float32),
        pltpu.SemaphoreType.DMA,
        pltpu.VMEM_SHARED((NPAD, D), jnp.float32),
    ],
)
def _sc_degrees(src_hbm, dst_hbm, out_hbm, src_v, dst_v, e0_v, e1_v, ss,
                deg_sh):
    c = lax.axis_index("c")
    s = lax.axis_index("s")
    t = c * 16 + s
    lane = lax.iota(jnp.int32, 16)
    one = jnp.full((16,), 1.0, jnp.float32)
    zero = jnp.full((16,), 0.0, jnp.float32)
    e0 = jnp.where(lane == 0, one, zero)
    e1 = jnp.where(lane == 1, one, zero)

    @pl.loop(0, K)
    def _(r):
        @pl.loop(0, D // 16)
        def _(k):
            e0_v[r, pl.ds(k * 16, 16)] = zero
            e1_v[r, pl.ds(k * 16, 16)] = zero

    @pl.loop(0, RPT // ZR)
    def _(z):
        pltpu.sync_copy(e0_v, deg_sh.at[pl.ds(s * RPT + z * ZR, ZR)])

    @pl.loop(0, K)
    def _(r):
        e0_v[r, pl.ds(0, 16)] = e0
        e1_v[r, pl.ds(0, 16)] = e1

    def _wait_pair():
        pltpu.make_async_copy(e0_v, deg_sh.at[pl.ds(0, K)], ss).wait()
        pltpu.make_async_copy(e1_v, deg_sh.at[pl.ds(0, K)], ss).wait()

    plsc.subcore_barrier()

    @pl.loop(0, SUP)
    def _(u):
        pltpu.sync_copy(src_hbm.at[pl.ds(t * CH + u * G, G)], src_v)
        pltpu.sync_copy(dst_hbm.at[pl.ds(t * CH + u * G, G)], dst_v)
        pltpu.async_copy(e0_v, deg_sh.at[src_v.at[0]], ss, add=True)
        pltpu.async_copy(e1_v, deg_sh.at[dst_v.at[0]], ss, add=True)

        @pl.loop(1, G)
        def _(j):
            pltpu.async_copy(e0_v, deg_sh.at[src_v.at[j]], ss, add=True)
            pltpu.async_copy(e1_v, deg_sh.at[dst_v.at[j]], ss, add=True)
            _wait_pair()

        _wait_pair()

    plsc.subcore_barrier()

    @pl.when(c == 0)
    def _():
        pltpu.sync_copy(deg_sh.at[pl.ds(s * RPT, RPT)],
                        out_hbm.at[0].at[pl.ds(s * RPT, RPT)])

    @pl.when(c == 1)
    def _():
        pltpu.sync_copy(deg_sh.at[pl.ds(s * RPT, RPT)],
                        out_hbm.at[1].at[pl.ds(s * RPT, RPT)])


@functools.partial(
    pl.kernel,
    mesh=_mesh,
    out_type=jax.ShapeDtypeStruct((2, NPAD, D), jnp.float32),
    scratch_types=[
        pltpu.VMEM((G, K), jnp.int32),
        pltpu.VMEM((G, K), jnp.int32),
        pltpu.VMEM((K, D), jnp.float32),
        pltpu.VMEM((K, D), jnp.float32),
        pltpu.SemaphoreType.DMA,
        pltpu.SemaphoreType.DMA,
        pltpu.VMEM_SHARED((NPAD, D), jnp.float32),
    ],
)
def _sc_aggregate(table_hbm, src_hbm, dst_hbm, out_hbm, src_v, dst_v, rows_a,
                  rows_b, sga, sgb, agg_sh):
    c = lax.axis_index("c")
    s = lax.axis_index("s")
    t = c * 16 + s
    zero = jnp.full((16,), 0.0, jnp.float32)

    @pl.loop(0, K)
    def _(r):
        @pl.loop(0, D // 16)
        def _(k):
            rows_a[r, pl.ds(k * 16, 16)] = zero

    @pl.loop(0, RPT // ZR)
    def _(z):
        pltpu.sync_copy(rows_a, agg_sh.at[pl.ds(s * RPT + z * ZR, ZR)])

    def _wait(buf, sem):
        pltpu.make_async_copy(table_hbm.at[pl.ds(0, K)], buf, sem).wait()

    plsc.subcore_barrier()

    @pl.loop(0, SUP)
    def _(u):
        pltpu.sync_copy(src_hbm.at[pl.ds(t * CH + u * G, G)], src_v)
        pltpu.sync_copy(dst_hbm.at[pl.ds(t * CH + u * G, G)], dst_v)
        pltpu.async_copy(table_hbm.at[src_v.at[0]], rows_a, sga)

        @pl.loop(0, G // 2)
        def _(jj):
            j = 2 * jj
            pltpu.async_copy(table_hbm.at[src_v.at[j + 1]], rows_b, sgb)
            _wait(rows_a, sga)
            pltpu.sync_copy(rows_a, agg_sh.at[dst_v.at[j]], add=True)

            @pl.when(j + 2 < G)
            def _():
                pltpu.async_copy(table_hbm.at[src_v.at[j + 2]], rows_a, sga)

            _wait(rows_b, sgb)
            pltpu.sync_copy(rows_b, agg_sh.at[dst_v.at[j + 1]], add=True)

    plsc.subcore_barrier()

    @pl.when(c == 0)
    def _():
        pltpu.sync_copy(agg_sh.at[pl.ds(s * RPT, RPT)],
                        out_hbm.at[0].at[pl.ds(s * RPT, RPT)])

    @pl.when(c == 1)
    def _():
        pltpu.sync_copy(agg_sh.at[pl.ds(s * RPT, RPT)],
                        out_hbm.at[1].at[pl.ds(s * RPT, RPT)])


def _prep_body(x_ref, deg_ref, table_ref, sc_ref):
    deg = deg_ref[0] + deg_ref[1]
    s_out = lax.rsqrt(jnp.maximum(deg[:, 0], 1.0))
    s_in = lax.rsqrt(jnp.maximum(deg[:, 1], 1.0))
    sc_ref[0, :] = s_in
    sc_ref[1, :] = s_out
    table_ref[...] = x_ref[...] * s_out[:, None]


def _tc_prep(x_pad, deg):
    return pl.pallas_call(
        _prep_body,
        out_shape=[
            jax.ShapeDtypeStruct((NPAD, D), jnp.float32),
            jax.ShapeDtypeStruct((2, NPAD), jnp.float32),
        ],
    )(x_pad, deg)


BR = 1024
NB = NPAD // BR


def _layer_body(agg_ref, sc_ref, w_ref, b_ref, out_ref):
    a = (agg_ref[0] + agg_ref[1]) * sc_ref[0][:, None]
    h = jnp.dot(a, w_ref[...], preferred_element_type=jnp.float32)
    h = jnp.maximum(h + b_ref[0][None, :], 0.0)
    out_ref[...] = h * sc_ref[1][:, None]


def _tc_layer(agg, scales, w, b2d):
    return pl.pallas_call(
        _layer_body,
        grid=(NB,),
        in_specs=[
            pl.BlockSpec((2, BR, D), lambda i: (0, i, 0)),
            pl.BlockSpec((2, BR), lambda i: (0, i)),
            pl.BlockSpec((D, D), lambda i: (0, 0)),
            pl.BlockSpec((1, D), lambda i: (0, 0)),
        ],
        out_specs=pl.BlockSpec((BR, D), lambda i: (i, 0)),
        out_shape=jax.ShapeDtypeStruct((NPAD, D), jnp.float32),
    )(agg, scales, w, b2d)


def _final_body(agg_ref, sc_ref, w_ref, b_ref, out_ref):
    i = pl.program_id(0)
    a = (agg_ref[0] + agg_ref[1]) * sc_ref[0][:, None]
    h = jnp.dot(a, w_ref[...], preferred_element_type=jnp.float32)
    h = jnp.maximum(h + b_ref[0][None, :], 0.0)
    rows = i * BR + lax.broadcasted_iota(jnp.int32, (BR, D), 0)
    h = jnp.where(rows < N, h, 0.0)
    part = jnp.sum(h.reshape(BR // 8, 8, D), axis=0)

    @pl.when(i == 0)
    def _():
        out_ref[...] = part

    @pl.when(i != 0)
    def _():
        out_ref[...] += part


def _tc_final(agg, scales, w, b2d):
    return pl.pallas_call(
        _final_body,
        grid=(NB,),
        in_specs=[
            pl.BlockSpec((2, BR, D), lambda i: (0, i, 0)),
            pl.BlockSpec((2, BR), lambda i: (0, i)),
            pl.BlockSpec((D, D), lambda i: (0, 0)),
            pl.BlockSpec((1, D), lambda i: (0, 0)),
        ],
        out_specs=pl.BlockSpec((8, D), lambda i: (0, 0)),
        out_shape=jax.ShapeDtypeStruct((8, D), jnp.float32),
    )(agg, scales, w, b2d)


def _head_body(sums_l_ref, sums_r_ref, wf1a_ref, wf1b_ref, bf1_ref, wf2_ref,
               bf2_ref, out_ref):
    suml = jnp.sum(sums_l_ref[...], axis=0, keepdims=True)
    sumr = jnp.sum(sums_r_ref[...], axis=0, keepdims=True)
    h = (jnp.dot(suml, wf1a_ref[...], preferred_element_type=jnp.float32)
         + jnp.dot(sumr, wf1b_ref[...], preferred_element_type=jnp.float32)
         + bf1_ref[...])
    h = jnp.maximum(h, 0.0)
    out_ref[...] = (jnp.sum(h * wf2_ref[...], axis=1, keepdims=True)
                    + bf2_ref[...])


def _tc_head(sums_l, sums_r, wf1a, wf1b, bf1_2d, wf2_t, bf2_2d):
    return pl.pallas_call(
        _head_body,
        out_shape=jax.ShapeDtypeStruct((1, 1), jnp.float32),
    )(sums_l, sums_r, wf1a, wf1b, bf1_2d, wf2_t, bf2_2d)


def _pad_edges(idx):
    pad = EPAD - E
    fill = N + (jnp.arange(pad, dtype=jnp.int32) % (NPAD - N))
    return jnp.concatenate([idx.astype(jnp.int32), fill]).reshape(
        32 * CH, K)


def kernel(xl, edge_index_l, xr, edge_index_r,
           W0, b0, W1, b1, W2, b2, Wf1, bf1, Wf2, bf2):
    sl, dl = _pad_edges(edge_index_l[0]), _pad_edges(edge_index_l[1])
    sr, dr = _pad_edges(edge_index_r[0]), _pad_edges(edge_index_r[1])

    xl_pad = jnp.pad(xl, ((0, NPAD - N), (0, 0)))
    xr_pad = jnp.pad(xr, ((0, NPAD - N), (0, 0)))

    tbl_l, sc_l = _tc_prep(xl_pad, _sc_degrees(sl, dl))
    tbl_r, sc_r = _tc_prep(xr_pad, _sc_degrees(sr, dr))

    for w, b in ((W0, b0), (W1, b1)):
        tbl_l = _tc_layer(_sc_aggregate(tbl_l, sl, dl), sc_l, w,
                          b.reshape(1, D))
        tbl_r = _tc_layer(_sc_aggregate(tbl_r, sr, dr), sc_r, w,
                          b.reshape(1, D))

    sums_l = _tc_final(_sc_aggregate(tbl_l, sl, dl), sc_l, W2,
                       b2.reshape(1, D))
    sums_r = _tc_final(_sc_aggregate(tbl_r, sr, dr), sc_r, W2,
                       b2.reshape(1, D))

    return _tc_head(sums_l, sums_r, Wf1[:D], Wf1[D:], bf1.reshape(1, D),
                    Wf2.reshape(1, D), bf2.reshape(1, 1))
